# baseline (speedup 1.0000x reference)
import functools

import jax
import jax.numpy as jnp
from jax import lax
from jax.experimental import pallas as pl
from jax.experimental.pallas import tpu as pltpu
from jax.experimental.pallas import tpu_sc as plsc

L = 16
NC = 1
NS = 16
NW = NC * NS


def _lane_gather(vec, idx):
    return jnp.take_along_axis(vec, idx, axis=0, mode="promise_in_bounds")


def kernel(source, target, emb, W, b):
    n = source.shape[0]
    rows = emb.shape[0]
    chunk = n // NW

    def _sc_body(src_hbm, tgt_hbm, emb_hbm, w_hbm, b_hbm, out_hbm,
                 src_v, tgt_v, out_v, emb_v, w_v, b_v, sem):
        wid = lax.axis_index("s") * NC + lax.axis_index("c")
        base = wid * chunk

        c1 = pltpu.async_copy(emb_hbm, emb_v, sem)
        c2 = pltpu.async_copy(w_hbm, w_v, sem)
        c3 = pltpu.async_copy(b_hbm, b_v.at[pl.ds(0, 1)], sem)
        c4 = pltpu.async_copy(src_hbm.at[pl.ds(base, chunk)], src_v, sem)
        c5 = pltpu.async_copy(tgt_hbm.at[pl.ds(base, chunk)], tgt_v, sem)
        c1.wait()
        c2.wait()
        c3.wait()
        c4.wait()
        c5.wait()

        w_reg = w_v[...]
        lane = lax.iota(jnp.int32, L)
        v_acc = jnp.zeros((L,), jnp.float32)
        for r in range(rows):
            s_r = jnp.sum(emb_v[r] * w_reg)
            v_acc = jnp.where(lane == r, s_r, v_acc)
        b_vec = jnp.full((L,), b_v[...][0])

        for i in range(chunk // L):
            s_idx = src_v[pl.ds(i * L, L)]
            t_idx = tgt_v[pl.ds(i * L, L)]
            vs = _lane_gather(v_acc, s_idx)
            vt = _lane_gather(v_acc, t_idx)
            out_v[pl.ds(i * L, L)] = vs + vt + b_vec

        pltpu.sync_copy(out_v, out_hbm.at[pl.ds(base, chunk)])

    mesh = plsc.VectorSubcoreMesh(
        core_axis_name="c", subcore_axis_name="s", num_cores=NC)
    k = functools.partial(
        pl.kernel,
        mesh=mesh,
        out_type=jax.ShapeDtypeStruct((n,), jnp.float32),
        compiler_params=pltpu.CompilerParams(needs_layout_passes=False),
        scratch_types=[
            pltpu.VMEM((chunk,), jnp.int32),
            pltpu.VMEM((chunk,), jnp.int32),
            pltpu.VMEM((chunk,), jnp.float32),
            pltpu.VMEM((rows, L), jnp.float32),
            pltpu.VMEM((L,), jnp.float32),
            pltpu.VMEM((L,), jnp.float32),
            pltpu.SemaphoreType.DMA,
        ],
    )(_sc_body)
    out = k(source.astype(jnp.int32), target.astype(jnp.int32),
            emb.astype(jnp.float32), W.astype(jnp.float32).reshape(L),
            b.astype(jnp.float32))
    return out.reshape(n, 1)


# speedup vs baseline: 1.0024x; 1.0024x over previous
import functools

import jax
import jax.numpy as jnp
from jax import lax
from jax.experimental import pallas as pl
from jax.experimental.pallas import tpu as pltpu
from jax.experimental.pallas import tpu_sc as plsc

L = 16
NC = 1
NS = 16
NW = NC * NS


def _lane_gather(vec, idx):
    return jnp.take_along_axis(vec, idx, axis=0, mode="promise_in_bounds")


def kernel(source, target, emb, W, b):
    n = source.shape[0]
    rows = emb.shape[0]
    chunk = n // NW

    def _sc_body(src_hbm, tgt_hbm, emb_hbm, w_hbm, b_hbm, out_hbm,
                 src_v, tgt_v, out_v, emb_v, w_v, b_v, sem):
        wid = lax.axis_index("s") * NC + lax.axis_index("c")
        base = wid * chunk

        c1 = pltpu.async_copy(emb_hbm, emb_v, sem)
        c2 = pltpu.async_copy(w_hbm, w_v, sem)
        c3 = pltpu.async_copy(b_hbm, b_v.at[pl.ds(0, 1)], sem)
        c4 = pltpu.async_copy(src_hbm.at[pl.ds(base, chunk)], src_v, sem)
        c5 = pltpu.async_copy(tgt_hbm.at[pl.ds(base, chunk)], tgt_v, sem)
        c1.wait()
        c2.wait()
        c3.wait()

        w_reg = w_v[...]
        lane = lax.iota(jnp.int32, L)
        v_acc = jnp.zeros((L,), jnp.float32)
        for r in range(rows):
            s_r = jnp.sum(emb_v[r] * w_reg)
            v_acc = jnp.where(lane == r, s_r, v_acc)
        b_vec = jnp.full((L,), b_v[...][0])

        c4.wait()
        c5.wait()

        half = chunk // (2 * L)
        for i in range(half):
            s_idx = src_v[pl.ds(i * L, L)]
            t_idx = tgt_v[pl.ds(i * L, L)]
            vs = _lane_gather(v_acc, s_idx)
            vt = _lane_gather(v_acc, t_idx)
            out_v[pl.ds(i * L, L)] = vs + vt + b_vec
        c6 = pltpu.async_copy(
            out_v.at[pl.ds(0, chunk // 2)],
            out_hbm.at[pl.ds(base, chunk // 2)], sem)
        for i in range(half, chunk // L):
            s_idx = src_v[pl.ds(i * L, L)]
            t_idx = tgt_v[pl.ds(i * L, L)]
            vs = _lane_gather(v_acc, s_idx)
            vt = _lane_gather(v_acc, t_idx)
            out_v[pl.ds(i * L, L)] = vs + vt + b_vec
        c7 = pltpu.async_copy(
            out_v.at[pl.ds(chunk // 2, chunk // 2)],
            out_hbm.at[pl.ds(base + chunk // 2, chunk // 2)], sem)
        c6.wait()
        c7.wait()

    mesh = plsc.VectorSubcoreMesh(
        core_axis_name="c", subcore_axis_name="s", num_cores=NC)
    k = functools.partial(
        pl.kernel,
        mesh=mesh,
        out_type=jax.ShapeDtypeStruct((n,), jnp.float32),
        compiler_params=pltpu.CompilerParams(needs_layout_passes=False),
        scratch_types=[
            pltpu.VMEM((chunk,), jnp.int32),
            pltpu.VMEM((chunk,), jnp.int32),
            pltpu.VMEM((chunk,), jnp.float32),
            pltpu.VMEM((rows, L), jnp.float32),
            pltpu.VMEM((L,), jnp.float32),
            pltpu.VMEM((L,), jnp.float32),
            pltpu.SemaphoreType.DMA,
        ],
    )(_sc_body)
    out = k(source.astype(jnp.int32), target.astype(jnp.int32),
            emb.astype(jnp.float32), W.astype(jnp.float32).reshape(L),
            b.astype(jnp.float32))
    return out.reshape(n, 1)
